# Initial kernel scaffold; baseline (speedup 1.0000x reference)
import jax
import jax.numpy as jnp
from jax.experimental import pallas as pl


def kernel(x, edge_index, edge_type, W1, root1, b1, W2, root2, b2):
    raise NotImplementedError("write your pallas kernel here")


# speedup vs baseline: 18.9235x; 18.9235x over previous
import functools

import jax
import jax.numpy as jnp
from jax import lax
from jax.experimental import pallas as pl
from jax.experimental.pallas import tpu as pltpu
from jax.experimental.pallas import tpu_sc as plsc

NC = 2
NS = 16
NW = NC * NS
LANES = 16
B = 80
CR = 25


def _make_counts_kernel(E, NP, R):
    EW = E // NW
    RW = EW // B
    NCH = RW // CR
    ST = NP // NS
    mesh = plsc.VectorSubcoreMesh(core_axis_name="c", subcore_axis_name="s")

    @functools.partial(
        pl.kernel,
        out_type=jax.ShapeDtypeStruct((NC * NP,), jnp.float32),
        mesh=mesh,
        compiler_params=pltpu.CompilerParams(needs_layout_passes=False),
        scratch_types=[
            pltpu.VMEM((CR, B), jnp.int32),
            pltpu.VMEM((CR, B), jnp.int32),
            pltpu.VMEM((CR, B), jnp.int32),
            pltpu.VMEM((B,), jnp.float32),
            pltpu.VMEM_SHARED((NP,), jnp.float32),
        ],
    )
    def counts_kernel(dst2_h, et2_h, zeros_h, cnt_h,
                      dstc, etc_, pairc, ones_v, cnt_sp):
        cid = lax.axis_index("c")
        sid = lax.axis_index("s")
        wid = sid * NC + cid
        for k in range(B // LANES):
            ones_v[pl.ds(k * LANES, LANES)] = jnp.ones((LANES,), jnp.float32)
        pltpu.sync_copy(zeros_h, cnt_sp.at[pl.ds(sid * ST, ST)])
        plsc.subcore_barrier()

        def chunk_body(c, carry):
            ch = wid * NCH + c
            pltpu.sync_copy(dst2_h.at[ch], dstc)
            pltpu.sync_copy(et2_h.at[ch], etc_)

            def sb_body(sb, carry2):
                for k in range(B // LANES):
                    sl = pl.ds(k * LANES, LANES)
                    d16 = dstc[sb, sl]
                    t16 = etc_[sb, sl]
                    pairc[sb, sl] = d16 * R + t16
                pltpu.sync_copy(ones_v, cnt_sp.at[pairc.at[sb]], add=True)
                return carry2

            return lax.fori_loop(0, CR, sb_body, carry)

        lax.fori_loop(0, NCH, chunk_body, 0)
        plsc.subcore_barrier()
        pltpu.sync_copy(cnt_sp.at[pl.ds(sid * ST, ST)],
                        cnt_h.at[pl.ds(cid * NP + sid * ST, ST)])

    return counts_kernel


def _make_agg_kernel(E, NN, D, NP, R, RS):
    EW = E // NW
    RW = EW // B
    NCH = RW // CR
    mesh = plsc.VectorSubcoreMesh(core_axis_name="c", subcore_axis_name="s")

    @functools.partial(
        pl.kernel,
        out_type=jax.ShapeDtypeStruct((NC * NS, RS, D), jnp.float32),
        mesh=mesh,
        compiler_params=pltpu.CompilerParams(needs_layout_passes=False),
        scratch_types=[
            pltpu.VMEM((CR, B), jnp.int32),
            pltpu.VMEM((CR, B), jnp.int32),
            pltpu.VMEM((CR, B), jnp.int32),
            pltpu.VMEM((CR, B), jnp.int32),
            pltpu.VMEM((CR, B), jnp.int32),
            pltpu.VMEM((B,), jnp.float32),
            pltpu.VMEM((B, D), jnp.float32),
            pltpu.VMEM_SHARED((NS * RS, D), jnp.float32),
            pltpu.SemaphoreType.DMA,
            pltpu.SemaphoreType.DMA,
        ],
    )
    def agg_kernel(h_h, src2_h, dst2_h, et2_h, norm_h, zrow_h, out_h,
                   srcc, dstc, etc_, keyc, pairc, scaleb, rows, acc_sp,
                   sem, sem2):
        cid = lax.axis_index("c")
        sid = lax.axis_index("s")
        wid = sid * NC + cid
        pltpu.sync_copy(zrow_h, acc_sp.at[pl.ds(sid * RS, RS)])
        plsc.subcore_barrier()

        def chunk_body(c, carry):
            ch = wid * NCH + c
            pltpu.sync_copy(src2_h.at[ch], srcc)
            pltpu.sync_copy(dst2_h.at[ch], dstc)
            pltpu.sync_copy(et2_h.at[ch], etc_)

            def sb_body(sb, carry2):
                for k in range(B // LANES):
                    sl = pl.ds(k * LANES, LANES)
                    s16 = srcc[sb, sl]
                    d16 = dstc[sb, sl]
                    t16 = etc_[sb, sl]
                    keyc[sb, sl] = t16 * NN + s16
                    pairc[sb, sl] = d16 * R + t16
                cp_rows = pltpu.async_copy(h_h.at[keyc.at[sb]], rows, sem)
                cp_sc = pltpu.async_copy(norm_h.at[pairc.at[sb]], scaleb, sem2)
                cp_rows.wait()
                cp_sc.wait()

                def row_body(r, carry3):
                    sc16 = plsc.load_gather(
                        scaleb, [lax.broadcast(r, (LANES,))])
                    for k in range(D // LANES):
                        sl = pl.ds(k * LANES, LANES)
                        rows[r, sl] = rows[r, sl] * sc16
                    return carry3

                lax.fori_loop(0, B, row_body, 0)
                pltpu.sync_copy(rows, acc_sp.at[dstc.at[sb]], add=True)
                return carry2

            return lax.fori_loop(0, CR, sb_body, carry)

        lax.fori_loop(0, NCH, chunk_body, 0)
        plsc.subcore_barrier()
        pltpu.sync_copy(acc_sp.at[pl.ds(sid * RS, RS)],
                        out_h.at[cid * NS + sid])

    return agg_kernel


def _norm_body(cnt_ref, norm_ref):
    c = cnt_ref[0] + cnt_ref[1]
    norm_ref[...] = 1.0 / jnp.maximum(c, 1.0)


def _norm_call(cnt_part, NP):
    rows = NP // 128
    cnt3 = cnt_part.reshape(NC, rows, 128)
    norm = pl.pallas_call(
        _norm_body,
        out_shape=jax.ShapeDtypeStruct((rows, 128), jnp.float32),
    )(cnt3)
    return norm.reshape(NP)


def _xform_body(x_ref, w_ref, root_ref, b_ref, h_ref, y_ref):
    xb = x_ref[...]
    for r in range(w_ref.shape[0]):
        h_ref[r] = jnp.dot(xb, w_ref[r], preferred_element_type=jnp.float32)
    y_ref[...] = (jnp.dot(xb, root_ref[...], preferred_element_type=jnp.float32)
                  + b_ref[...])


def _xform_call(x, W, root, b, BN=1000):
    NN, D = x.shape
    R = W.shape[0]
    grid = NN // BN
    h, y = pl.pallas_call(
        _xform_body,
        grid=(grid,),
        in_specs=[
            pl.BlockSpec((BN, D), lambda i: (i, 0)),
            pl.BlockSpec((R, D, D), lambda i: (0, 0, 0)),
            pl.BlockSpec((D, D), lambda i: (0, 0)),
            pl.BlockSpec((1, D), lambda i: (0, 0)),
        ],
        out_specs=[
            pl.BlockSpec((R, BN, D), lambda i: (0, i, 0)),
            pl.BlockSpec((BN, D), lambda i: (i, 0)),
        ],
        out_shape=[
            jax.ShapeDtypeStruct((R, NN, D), jnp.float32),
            jax.ShapeDtypeStruct((NN, D), jnp.float32),
        ],
    )(x, W, root, b.reshape(1, D))
    return h.reshape(R * NN, D), y


def _combine_xform_body(p_ref, yr_ref, w_ref, root_ref, b_ref, h_ref, y_ref):
    xb = jnp.maximum(p_ref[0] + p_ref[1] + yr_ref[...], 0.0)
    for r in range(w_ref.shape[0]):
        h_ref[r] = jnp.dot(xb, w_ref[r], preferred_element_type=jnp.float32)
    y_ref[...] = (jnp.dot(xb, root_ref[...], preferred_element_type=jnp.float32)
                  + b_ref[...])


def _combine_xform_call(p, yr, W, root, b, BN=1000):
    NN, D = yr.shape
    R = W.shape[0]
    grid = NN // BN
    p3 = p.reshape(NC, NN, D)
    h, y = pl.pallas_call(
        _combine_xform_body,
        grid=(grid,),
        in_specs=[
            pl.BlockSpec((NC, BN, D), lambda i: (0, i, 0)),
            pl.BlockSpec((BN, D), lambda i: (i, 0)),
            pl.BlockSpec((R, D, D), lambda i: (0, 0, 0)),
            pl.BlockSpec((D, D), lambda i: (0, 0)),
            pl.BlockSpec((1, D), lambda i: (0, 0)),
        ],
        out_specs=[
            pl.BlockSpec((R, BN, D), lambda i: (0, i, 0)),
            pl.BlockSpec((BN, D), lambda i: (i, 0)),
        ],
        out_shape=[
            jax.ShapeDtypeStruct((R, NN, D), jnp.float32),
            jax.ShapeDtypeStruct((NN, D), jnp.float32),
        ],
    )(p3, yr, W, root, b.reshape(1, D))
    return h.reshape(R * NN, D), y


def _final_body(p_ref, yr_ref, out_ref):
    out_ref[...] = p_ref[0] + p_ref[1] + yr_ref[...]


def _final_call(p, yr, BN=1000):
    NN, D = yr.shape
    grid = NN // BN
    p3 = p.reshape(NC, NN, D)
    return pl.pallas_call(
        _final_body,
        grid=(grid,),
        in_specs=[
            pl.BlockSpec((NC, BN, D), lambda i: (0, i, 0)),
            pl.BlockSpec((BN, D), lambda i: (i, 0)),
        ],
        out_specs=pl.BlockSpec((BN, D), lambda i: (i, 0)),
        out_shape=jax.ShapeDtypeStruct((NN, D), jnp.float32),
    )(p3, yr)


def kernel(x, edge_index, edge_type, W1, root1, b1, W2, root2, b2):
    NN, D = x.shape
    R = W1.shape[0]
    E = edge_type.shape[0]
    NP = NN * R
    NP2 = ((NP + NS * 128 - 1) // (NS * 128)) * (NS * 128)
    RS2 = ((NN // NS) + 7) // 8 * 8
    NN2 = NS * RS2

    src2 = edge_index[0].reshape(E // (CR * B), CR, B)
    dst2 = edge_index[1].reshape(E // (CR * B), CR, B)
    et2 = edge_type.reshape(E // (CR * B), CR, B)
    zeros_cnt = jnp.zeros((NP2 // NS,), jnp.float32)
    zeros_row = jnp.zeros((RS2, D), jnp.float32)

    cnt_part = _make_counts_kernel(E, NP2, R)(dst2, et2, zeros_cnt)
    norm = _norm_call(cnt_part, NP2)

    agg = _make_agg_kernel(E, NN, D, NP2, R, RS2)

    h1, yr1 = _xform_call(x, W1, root1, b1)
    p1 = agg(h1, src2, dst2, et2, norm, zeros_row)
    p1 = p1.reshape(NC, NN2, D)[:, :NN]
    h2, yr2 = _combine_xform_call(p1, yr1, W2, root2, b2)
    p2 = agg(h2, src2, dst2, et2, norm, zeros_row)
    p2 = p2.reshape(NC, NN2, D)[:, :NN]
    return _final_call(p2, yr2)


# speedup vs baseline: 28.8342x; 1.5237x over previous
import functools

import jax
import jax.numpy as jnp
from jax import lax
from jax.experimental import pallas as pl
from jax.experimental.pallas import tpu as pltpu
from jax.experimental.pallas import tpu_sc as plsc

NC = 2
NS = 16
NW = NC * NS
LANES = 16
B = 80
CR = 25


def _make_counts_kernel(E, NP, R):
    EW = E // NW
    RW = EW // B
    NCH = RW // CR
    ST = NP // NS
    mesh = plsc.VectorSubcoreMesh(core_axis_name="c", subcore_axis_name="s")

    @functools.partial(
        pl.kernel,
        out_type=jax.ShapeDtypeStruct((NC * NP,), jnp.float32),
        mesh=mesh,
        compiler_params=pltpu.CompilerParams(needs_layout_passes=False),
        scratch_types=[
            pltpu.VMEM((CR, B), jnp.int32),
            pltpu.VMEM((CR, B), jnp.int32),
            pltpu.VMEM((CR, B), jnp.int32),
            pltpu.VMEM((B,), jnp.float32),
            pltpu.VMEM_SHARED((NP,), jnp.float32),
        ],
    )
    def counts_kernel(dst2_h, et2_h, zeros_h, cnt_h,
                      dstc, etc_, pairc, ones_v, cnt_sp):
        cid = lax.axis_index("c")
        sid = lax.axis_index("s")
        wid = sid * NC + cid
        for k in range(B // LANES):
            ones_v[pl.ds(k * LANES, LANES)] = jnp.ones((LANES,), jnp.float32)
        pltpu.sync_copy(zeros_h, cnt_sp.at[pl.ds(sid * ST, ST)])
        plsc.subcore_barrier()

        def chunk_body(c, carry):
            ch = wid * NCH + c
            pltpu.sync_copy(dst2_h.at[ch], dstc)
            pltpu.sync_copy(et2_h.at[ch], etc_)

            def sb_body(sb, carry2):
                for k in range(B // LANES):
                    sl = pl.ds(k * LANES, LANES)
                    d16 = dstc[sb, sl]
                    t16 = etc_[sb, sl]
                    pairc[sb, sl] = d16 * R + t16
                pltpu.sync_copy(ones_v, cnt_sp.at[pairc.at[sb]], add=True)
                return carry2

            return lax.fori_loop(0, CR, sb_body, carry)

        lax.fori_loop(0, NCH, chunk_body, 0)
        plsc.subcore_barrier()
        pltpu.sync_copy(cnt_sp.at[pl.ds(sid * ST, ST)],
                        cnt_h.at[pl.ds(cid * NP + sid * ST, ST)])

    return counts_kernel


def _make_agg_kernel(E, NN, D, NP, R, RS):
    EW = E // NW
    RW = EW // B
    NCH = RW // CR
    mesh = plsc.VectorSubcoreMesh(core_axis_name="c", subcore_axis_name="s")

    @functools.partial(
        pl.kernel,
        out_type=jax.ShapeDtypeStruct((NC * NS, RS, D), jnp.float32),
        mesh=mesh,
        compiler_params=pltpu.CompilerParams(needs_layout_passes=False),
        scratch_types=[
            pltpu.VMEM((CR, B), jnp.int32),
            pltpu.VMEM((CR, B), jnp.int32),
            pltpu.VMEM((CR, B), jnp.int32),
            pltpu.VMEM((CR, B), jnp.int32),
            pltpu.VMEM((CR, B), jnp.int32),
            pltpu.VMEM((B,), jnp.float32),
            pltpu.VMEM((B,), jnp.float32),
            pltpu.VMEM((B, D), jnp.float32),
            pltpu.VMEM((B, D), jnp.float32),
            pltpu.VMEM_SHARED((NS * RS, D), jnp.float32),
            pltpu.SemaphoreType.DMA,
            pltpu.SemaphoreType.DMA,
            pltpu.SemaphoreType.DMA,
            pltpu.SemaphoreType.DMA,
        ],
    )
    def agg_kernel(h_h, src2_h, dst2_h, et2_h, norm_h, zrow_h, out_h,
                   srcc, dstc, etc_, keyc, pairc, sc0, sc1, rows0, rows1,
                   acc_sp, semr0, semr1, sems0, sems1):
        cid = lax.axis_index("c")
        sid = lax.axis_index("s")
        wid = sid * NC + cid
        pltpu.sync_copy(zrow_h, acc_sp.at[pl.ds(sid * RS, RS)])
        plsc.subcore_barrier()

        bufs = ((rows0, sc0, semr0, sems0), (rows1, sc1, semr1, sems1))

        def issue(sb, p):
            rows, scl, semr, sems = bufs[p]
            pltpu.async_copy(h_h.at[keyc.at[sb]], rows, semr)
            pltpu.async_copy(norm_h.at[pairc.at[sb]], scl, sems)

        def drain(sb, p):
            rows, scl, semr, sems = bufs[p]
            pltpu.make_async_copy(h_h.at[keyc.at[sb]], rows, semr).wait()
            pltpu.make_async_copy(norm_h.at[pairc.at[sb]], scl, sems).wait()

        def scale_scatter(sb, p):
            rows, scl, _, _ = bufs[p]

            def row_body(r, carry3):
                sc16 = plsc.load_gather(scl, [lax.broadcast(r, (LANES,))])
                for k in range(D // LANES):
                    sl = pl.ds(k * LANES, LANES)
                    rows[r, sl] = rows[r, sl] * sc16
                return carry3

            lax.fori_loop(0, B, row_body, 0)
            pltpu.sync_copy(rows, acc_sp.at[dstc.at[sb]], add=True)

        def chunk_body(c, carry):
            ch = wid * NCH + c
            pltpu.sync_copy(src2_h.at[ch], srcc)
            pltpu.sync_copy(dst2_h.at[ch], dstc)
            pltpu.sync_copy(et2_h.at[ch], etc_)

            def key_body(sb, carry2):
                for k in range(B // LANES):
                    sl = pl.ds(k * LANES, LANES)
                    s16 = srcc[sb, sl]
                    d16 = dstc[sb, sl]
                    t16 = etc_[sb, sl]
                    keyc[sb, sl] = t16 * NN + s16
                    pairc[sb, sl] = d16 * R + t16
                return carry2

            lax.fori_loop(0, CR, key_body, 0)

            issue(0, 0)

            def pair_body(i, carry2):
                sb = 2 * i
                drain(sb, 0)
                issue(sb + 1, 1)
                scale_scatter(sb, 0)
                drain(sb + 1, 1)
                issue(sb + 2, 0)
                scale_scatter(sb + 1, 1)
                return carry2

            lax.fori_loop(0, (CR - 1) // 2, pair_body, 0)
            drain(CR - 1, 0)
            scale_scatter(CR - 1, 0)
            return carry

        lax.fori_loop(0, NCH, chunk_body, 0)
        plsc.subcore_barrier()
        pltpu.sync_copy(acc_sp.at[pl.ds(sid * RS, RS)],
                        out_h.at[cid * NS + sid])

    return agg_kernel


def _norm_body(cnt_ref, norm_ref):
    c = cnt_ref[0] + cnt_ref[1]
    norm_ref[...] = 1.0 / jnp.maximum(c, 1.0)


def _norm_call(cnt_part, NP):
    rows = NP // 128
    cnt3 = cnt_part.reshape(NC, rows, 128)
    norm = pl.pallas_call(
        _norm_body,
        out_shape=jax.ShapeDtypeStruct((rows, 128), jnp.float32),
    )(cnt3)
    return norm.reshape(NP)


def _xform_body(x_ref, w_ref, root_ref, b_ref, h_ref, y_ref):
    xb = x_ref[...]
    for r in range(w_ref.shape[0]):
        h_ref[r] = jnp.dot(xb, w_ref[r], preferred_element_type=jnp.float32)
    y_ref[...] = (jnp.dot(xb, root_ref[...], preferred_element_type=jnp.float32)
                  + b_ref[...])


def _xform_call(x, W, root, b, BN=1000):
    NN, D = x.shape
    R = W.shape[0]
    grid = NN // BN
    h, y = pl.pallas_call(
        _xform_body,
        grid=(grid,),
        in_specs=[
            pl.BlockSpec((BN, D), lambda i: (i, 0)),
            pl.BlockSpec((R, D, D), lambda i: (0, 0, 0)),
            pl.BlockSpec((D, D), lambda i: (0, 0)),
            pl.BlockSpec((1, D), lambda i: (0, 0)),
        ],
        out_specs=[
            pl.BlockSpec((R, BN, D), lambda i: (0, i, 0)),
            pl.BlockSpec((BN, D), lambda i: (i, 0)),
        ],
        out_shape=[
            jax.ShapeDtypeStruct((R, NN, D), jnp.float32),
            jax.ShapeDtypeStruct((NN, D), jnp.float32),
        ],
    )(x, W, root, b.reshape(1, D))
    return h.reshape(R * NN, D), y


def _combine_xform_body(p_ref, yr_ref, w_ref, root_ref, b_ref, h_ref, y_ref):
    xb = jnp.maximum(p_ref[0] + p_ref[1] + yr_ref[...], 0.0)
    for r in range(w_ref.shape[0]):
        h_ref[r] = jnp.dot(xb, w_ref[r], preferred_element_type=jnp.float32)
    y_ref[...] = (jnp.dot(xb, root_ref[...], preferred_element_type=jnp.float32)
                  + b_ref[...])


def _combine_xform_call(p, yr, W, root, b, BN=1000):
    NN, D = yr.shape
    R = W.shape[0]
    grid = NN // BN
    p3 = p.reshape(NC, NN, D)
    h, y = pl.pallas_call(
        _combine_xform_body,
        grid=(grid,),
        in_specs=[
            pl.BlockSpec((NC, BN, D), lambda i: (0, i, 0)),
            pl.BlockSpec((BN, D), lambda i: (i, 0)),
            pl.BlockSpec((R, D, D), lambda i: (0, 0, 0)),
            pl.BlockSpec((D, D), lambda i: (0, 0)),
            pl.BlockSpec((1, D), lambda i: (0, 0)),
        ],
        out_specs=[
            pl.BlockSpec((R, BN, D), lambda i: (0, i, 0)),
            pl.BlockSpec((BN, D), lambda i: (i, 0)),
        ],
        out_shape=[
            jax.ShapeDtypeStruct((R, NN, D), jnp.float32),
            jax.ShapeDtypeStruct((NN, D), jnp.float32),
        ],
    )(p3, yr, W, root, b.reshape(1, D))
    return h.reshape(R * NN, D), y


def _final_body(p_ref, yr_ref, out_ref):
    out_ref[...] = p_ref[0] + p_ref[1] + yr_ref[...]


def _final_call(p, yr, BN=1000):
    NN, D = yr.shape
    grid = NN // BN
    p3 = p.reshape(NC, NN, D)
    return pl.pallas_call(
        _final_body,
        grid=(grid,),
        in_specs=[
            pl.BlockSpec((NC, BN, D), lambda i: (0, i, 0)),
            pl.BlockSpec((BN, D), lambda i: (i, 0)),
        ],
        out_specs=pl.BlockSpec((BN, D), lambda i: (i, 0)),
        out_shape=jax.ShapeDtypeStruct((NN, D), jnp.float32),
    )(p3, yr)


def kernel(x, edge_index, edge_type, W1, root1, b1, W2, root2, b2):
    NN, D = x.shape
    R = W1.shape[0]
    E = edge_type.shape[0]
    NP = NN * R
    NP2 = ((NP + NS * 128 - 1) // (NS * 128)) * (NS * 128)
    RS2 = ((NN // NS) + 7) // 8 * 8
    NN2 = NS * RS2

    src2 = edge_index[0].reshape(E // (CR * B), CR, B)
    dst2 = edge_index[1].reshape(E // (CR * B), CR, B)
    et2 = edge_type.reshape(E // (CR * B), CR, B)
    zeros_cnt = jnp.zeros((NP2 // NS,), jnp.float32)
    zeros_row = jnp.zeros((RS2, D), jnp.float32)

    cnt_part = _make_counts_kernel(E, NP2, R)(dst2, et2, zeros_cnt)
    norm = _norm_call(cnt_part, NP2)

    agg = _make_agg_kernel(E, NN, D, NP2, R, RS2)

    h1, yr1 = _xform_call(x, W1, root1, b1)
    p1 = agg(h1, src2, dst2, et2, norm, zeros_row)
    p1 = p1.reshape(NC, NN2, D)[:, :NN]
    h2, yr2 = _combine_xform_call(p1, yr1, W2, root2, b2)
    p2 = agg(h2, src2, dst2, et2, norm, zeros_row)
    p2 = p2.reshape(NC, NN2, D)[:, :NN]
    return _final_call(p2, yr2)
